# baseline (speedup 1.0000x reference)
import jax
import jax.numpy as jnp
from jax import lax
from jax.experimental import pallas as pl
from jax.experimental.pallas import tpu as pltpu
from jax.experimental.pallas import tpu_sc as plsc

B, T, D = 4096, 200, 64
NC, NS = 2, 16
NW = NC * NS
BPW = B // NW

N1 = 24 * 7
N2 = 32 * 13

NGROUPS = T // 16 + 1


def _sc_body(h_hbm, w_hbm, d_hbm, m_hbm,
             eh_hbm, ew_hbm, ed_hbm, em_hbm,
             out_hbm,
             eh_v, ew_v, ed_v, em_v,
             t1_v, t2_v,
             h0, w0, d0, m0, h1, w1, d1, m1,
             out0, out1,
             semi0, semi1, semo0, semo1, semt):
    wid = lax.axis_index("s") * NC + lax.axis_index("c")
    base = wid * BPW

    idx_srcs = (h_hbm, w_hbm, d_hbm, m_hbm)
    idx_bufs = ((h0, w0, d0, m0), (h1, w1, d1, m1))
    outs = (out0, out1)
    semis = (semi0, semi1)
    semos = (semo0, semo1)

    def start_idx(g, p):
        for src, dst in zip(idx_srcs, idx_bufs[p]):
            pltpu.async_copy(src.at[base + g], dst, semis[p])

    def wait_idx(p):
        for src, dst in zip(idx_srcs, idx_bufs[p]):
            pltpu.make_async_copy(src.at[0], dst, semis[p]).wait()

    start_idx(0, 0)

    for src, dst in zip((eh_hbm, ew_hbm, ed_hbm, em_hbm),
                        (eh_v, ew_v, ed_v, em_v)):
        pltpu.async_copy(src, dst, semt)
    for src, dst in zip((eh_hbm, ew_hbm, ed_hbm, em_hbm),
                        (eh_v, ew_v, ed_v, em_v)):
        pltpu.make_async_copy(src, dst, semt).wait()

    def build1(k, _):
        h = k // 7
        w = k - h * 7
        for j in range(D // 16):
            s = pl.ds(16 * j, 16)
            t1_v[pl.ds(k * D + 16 * j, 16)] = eh_v[h, s] + ew_v[w, s]
        return _

    lax.fori_loop(0, N1, build1, None)

    def build2(k, _):
        d = k // 13
        m = k - d * 13
        for j in range(D // 16):
            s = pl.ds(16 * j, 16)
            t2_v[pl.ds(k * D + 16 * j, 16)] = ed_v[d, s] + em_v[m, s]
        return _

    lax.fori_loop(0, N2, build2, None)

    def outer(gg, _):
        for p in range(2):
            g = gg * 2 + p

            @pl.when(g + 1 < BPW)
            def _prefetch():
                start_idx(g + 1, 1 - p)

            wait_idx(p)

            @pl.when(g >= 2)
            def _reclaim():
                pltpu.make_async_copy(outs[p], out_hbm.at[0], semos[p]).wait()

            hv, wv, dv, mv = idx_bufs[p]
            ov = outs[p]

            def group(q, c):
                qb = jnp.minimum(q * 16, T - 16)
                s = pl.ds(qb, 16)
                pk = ((hv[s] * 7 + wv[s]) * (D << 15)
                      + (dv[s] * 13 + mv[s]) * D)
                for l in range(16):
                    w = pk[l]
                    a = lax.shift_right_logical(w, 15)
                    b = lax.bitwise_and(w, 32767)
                    t = qb + l
                    for j in range(D // 16):
                        sj = pl.ds(16 * j, 16)
                        ov[t, sj] = (t1_v[pl.ds(a + 16 * j, 16)]
                                     + t2_v[pl.ds(b + 16 * j, 16)])
                return c

            lax.fori_loop(0, NGROUPS, group, None)

            pltpu.async_copy(ov, out_hbm.at[base + g], semos[p])
        return _

    lax.fori_loop(0, BPW // 2, outer, None)

    for p in range(2):
        pltpu.make_async_copy(outs[p], out_hbm.at[0], semos[p]).wait()


@jax.jit
def kernel(hour, weekday, day, month, E_hour, E_weekday, E_day, E_month):
    mesh = plsc.VectorSubcoreMesh(core_axis_name="c", subcore_axis_name="s")
    run = pl.kernel(
        _sc_body,
        out_type=jax.ShapeDtypeStruct((B, T, D), jnp.float32),
        mesh=mesh,
        compiler_params=pltpu.CompilerParams(use_tc_tiling_on_sc=True),
        scratch_types=[
            pltpu.VMEM((24, D), jnp.float32),
            pltpu.VMEM((7, D), jnp.float32),
            pltpu.VMEM((32, D), jnp.float32),
            pltpu.VMEM((13, D), jnp.float32),
            pltpu.VMEM((N1 * D,), jnp.float32),
            pltpu.VMEM((N2 * D,), jnp.float32),
            pltpu.VMEM((T,), jnp.int32),
            pltpu.VMEM((T,), jnp.int32),
            pltpu.VMEM((T,), jnp.int32),
            pltpu.VMEM((T,), jnp.int32),
            pltpu.VMEM((T,), jnp.int32),
            pltpu.VMEM((T,), jnp.int32),
            pltpu.VMEM((T,), jnp.int32),
            pltpu.VMEM((T,), jnp.int32),
            pltpu.VMEM((T, D), jnp.float32),
            pltpu.VMEM((T, D), jnp.float32),
            pltpu.SemaphoreType.DMA,
            pltpu.SemaphoreType.DMA,
            pltpu.SemaphoreType.DMA,
            pltpu.SemaphoreType.DMA,
            pltpu.SemaphoreType.DMA,
        ],
    )
    return run(hour, weekday, day, month, E_hour, E_weekday, E_day, E_month)


# speedup vs baseline: 1.2881x; 1.2881x over previous
import jax
import jax.numpy as jnp
from jax import lax
from jax.experimental import pallas as pl
from jax.experimental.pallas import tpu as pltpu
from jax.experimental.pallas import tpu_sc as plsc

B, T, D = 4096, 200, 64
NC, NS = 2, 16
NW = NC * NS
BPW = B // NW

N1 = 24 * 7
N2 = 32 * 13

NGROUPS = T // 16 + 1


def _sc_body(h_hbm, w_hbm, d_hbm, m_hbm,
             eh_hbm, ew_hbm, ed_hbm, em_hbm,
             out_hbm,
             eh_v, ew_v, ed_v, em_v,
             t1_v, t2_v,
             h0, w0, d0, m0, h1, w1, d1, m1,
             out0, out1,
             semi0, semi1, semo0, semo1, semt):
    wid = lax.axis_index("s") * NC + lax.axis_index("c")
    base = wid * BPW

    idx_srcs = (h_hbm, w_hbm, d_hbm, m_hbm)
    idx_bufs = ((h0, w0, d0, m0), (h1, w1, d1, m1))
    outs = (out0, out1)
    semis = (semi0, semi1)
    semos = (semo0, semo1)

    def start_idx(g, p):
        for src, dst in zip(idx_srcs, idx_bufs[p]):
            pltpu.async_copy(src.at[base + g], dst, semis[p])

    def wait_idx(p):
        for src, dst in zip(idx_srcs, idx_bufs[p]):
            pltpu.make_async_copy(src.at[0], dst, semis[p]).wait()

    start_idx(0, 0)

    for src, dst in zip((eh_hbm, ew_hbm, ed_hbm, em_hbm),
                        (eh_v, ew_v, ed_v, em_v)):
        pltpu.async_copy(src, dst, semt)
    for src, dst in zip((eh_hbm, ew_hbm, ed_hbm, em_hbm),
                        (eh_v, ew_v, ed_v, em_v)):
        pltpu.make_async_copy(src, dst, semt).wait()

    def build1(k, _):
        h = k // 7
        w = k - h * 7
        for j in range(D // 16):
            s = pl.ds(16 * j, 16)
            t1_v[pl.ds(k * D + 16 * j, 16)] = eh_v[h, s] + ew_v[w, s]
        return _

    lax.fori_loop(0, N1, build1, None)

    def build2(k, _):
        d = k // 13
        m = k - d * 13
        for j in range(D // 16):
            s = pl.ds(16 * j, 16)
            t2_v[pl.ds(k * D + 16 * j, 16)] = ed_v[d, s] + em_v[m, s]
        return _

    lax.fori_loop(0, N2, build2, None)

    def outer(gg, _):
        for p in range(2):
            g = gg * 2 + p

            @pl.when(g + 1 < BPW)
            def _prefetch():
                start_idx(g + 1, 1 - p)

            wait_idx(p)

            @pl.when(g >= 2)
            def _reclaim():
                pltpu.make_async_copy(outs[p], out_hbm.at[0], semos[p]).wait()

            hv, wv, dv, mv = idx_bufs[p]
            ov = outs[p]

            @plsc.parallel_loop(0, NGROUPS, unroll=2)
            def group(q):
                qb = jnp.minimum(q * 16, T - 16)
                s = pl.ds(qb, 16)
                pk = ((hv[s] * 7 + wv[s]) * (D << 15)
                      + (dv[s] * 13 + mv[s]) * D)
                for l in range(16):
                    w = pk[l]
                    a = lax.shift_right_logical(w, 15)
                    b = lax.bitwise_and(w, 32767)
                    t = qb + l
                    for j in range(D // 16):
                        sj = pl.ds(16 * j, 16)
                        ov[t, sj] = (t1_v[pl.ds(a + 16 * j, 16)]
                                     + t2_v[pl.ds(b + 16 * j, 16)])

            pltpu.async_copy(ov, out_hbm.at[base + g], semos[p])
        return _

    lax.fori_loop(0, BPW // 2, outer, None)

    for p in range(2):
        pltpu.make_async_copy(outs[p], out_hbm.at[0], semos[p]).wait()


@jax.jit
def kernel(hour, weekday, day, month, E_hour, E_weekday, E_day, E_month):
    mesh = plsc.VectorSubcoreMesh(core_axis_name="c", subcore_axis_name="s")
    run = pl.kernel(
        _sc_body,
        out_type=jax.ShapeDtypeStruct((B, T, D), jnp.float32),
        mesh=mesh,
        compiler_params=pltpu.CompilerParams(use_tc_tiling_on_sc=True),
        scratch_types=[
            pltpu.VMEM((24, D), jnp.float32),
            pltpu.VMEM((7, D), jnp.float32),
            pltpu.VMEM((32, D), jnp.float32),
            pltpu.VMEM((13, D), jnp.float32),
            pltpu.VMEM((N1 * D,), jnp.float32),
            pltpu.VMEM((N2 * D,), jnp.float32),
            pltpu.VMEM((T,), jnp.int32),
            pltpu.VMEM((T,), jnp.int32),
            pltpu.VMEM((T,), jnp.int32),
            pltpu.VMEM((T,), jnp.int32),
            pltpu.VMEM((T,), jnp.int32),
            pltpu.VMEM((T,), jnp.int32),
            pltpu.VMEM((T,), jnp.int32),
            pltpu.VMEM((T,), jnp.int32),
            pltpu.VMEM((T, D), jnp.float32),
            pltpu.VMEM((T, D), jnp.float32),
            pltpu.SemaphoreType.DMA,
            pltpu.SemaphoreType.DMA,
            pltpu.SemaphoreType.DMA,
            pltpu.SemaphoreType.DMA,
            pltpu.SemaphoreType.DMA,
        ],
    )
    return run(hour, weekday, day, month, E_hour, E_weekday, E_day, E_month)


# speedup vs baseline: 1.5592x; 1.2104x over previous
import jax
import jax.numpy as jnp
from jax import lax
from jax.experimental import pallas as pl
from jax.experimental.pallas import tpu as pltpu
from jax.experimental.pallas import tpu_sc as plsc

B, T, D = 4096, 200, 64
NC, NS = 2, 16
NW = NC * NS
BPW = B // NW

N1 = 24 * 7
N2 = 32 * 13

NGROUPS = T // 16 + 1


def _sc_body(h_hbm, w_hbm, d_hbm, m_hbm,
             eh_hbm, ew_hbm, ed_hbm, em_hbm,
             out_hbm,
             eh_v, ew_v, ed_v, em_v,
             t1_v, t2_v,
             h0, w0, d0, m0, h1, w1, d1, m1,
             out0, out1,
             semi0, semi1, semo0, semo1, semt):
    wid = lax.axis_index("s") * NC + lax.axis_index("c")
    base = wid * BPW

    idx_srcs = (h_hbm, w_hbm, d_hbm, m_hbm)
    idx_bufs = ((h0, w0, d0, m0), (h1, w1, d1, m1))
    outs = (out0, out1)
    semis = (semi0, semi1)
    semos = (semo0, semo1)

    def start_idx(g, p):
        for src, dst in zip(idx_srcs, idx_bufs[p]):
            pltpu.async_copy(src.at[base + g], dst, semis[p])

    def wait_idx(p):
        for src, dst in zip(idx_srcs, idx_bufs[p]):
            pltpu.make_async_copy(src.at[0], dst, semis[p]).wait()

    start_idx(0, 0)

    for src, dst in zip((eh_hbm, ew_hbm, ed_hbm, em_hbm),
                        (eh_v, ew_v, ed_v, em_v)):
        pltpu.async_copy(src, dst, semt)
    for src, dst in zip((eh_hbm, ew_hbm, ed_hbm, em_hbm),
                        (eh_v, ew_v, ed_v, em_v)):
        pltpu.make_async_copy(src, dst, semt).wait()

    def build1(k, _):
        h = k // 7
        w = k - h * 7
        for half in range(2):
            lo = eh_v[h, pl.ds(32 * half, 16)] + ew_v[w, pl.ds(32 * half, 16)]
            hi = (eh_v[h, pl.ds(32 * half + 16, 16)]
                  + ew_v[w, pl.ds(32 * half + 16, 16)])
            pk = plsc.pack(lo, hi, format=plsc.PackFormat.INTERLEAVED)
            t1_v[pl.ds(k * 32 + 16 * half, 16)] = plsc.bitcast(pk, jnp.int32)
        return _

    lax.fori_loop(0, N1, build1, None)

    def build2(k, _):
        d = k // 13
        m = k - d * 13
        for half in range(2):
            lo = ed_v[d, pl.ds(32 * half, 16)] + em_v[m, pl.ds(32 * half, 16)]
            hi = (ed_v[d, pl.ds(32 * half + 16, 16)]
                  + em_v[m, pl.ds(32 * half + 16, 16)])
            pk = plsc.pack(lo, hi, format=plsc.PackFormat.INTERLEAVED)
            t2_v[pl.ds(k * 32 + 16 * half, 16)] = plsc.bitcast(pk, jnp.int32)
        return _

    lax.fori_loop(0, N2, build2, None)

    def outer(gg, _):
        for p in range(2):
            g = gg * 2 + p

            @pl.when(g + 1 < BPW)
            def _prefetch():
                start_idx(g + 1, 1 - p)

            wait_idx(p)

            @pl.when(g >= 2)
            def _reclaim():
                pltpu.make_async_copy(outs[p], out_hbm.at[0], semos[p]).wait()

            hv, wv, dv, mv = idx_bufs[p]
            ov = outs[p]

            @plsc.parallel_loop(0, NGROUPS, unroll=4)
            def group(q):
                qb = jnp.minimum(q * 16, T - 16)
                s = pl.ds(qb, 16)
                pk = ((hv[s] * 7 + wv[s]) * (32 << 15)
                      + (dv[s] * 13 + mv[s]) * 32)
                for l in range(16):
                    w = pk[l]
                    a = lax.shift_right_logical(w, 15)
                    b = lax.bitwise_and(w, 32767)
                    t = qb + l
                    for half in range(2):
                        u = plsc.bitcast(t1_v[pl.ds(a + 16 * half, 16)],
                                         jnp.bfloat16)
                        v = plsc.bitcast(t2_v[pl.ds(b + 16 * half, 16)],
                                         jnp.bfloat16)
                        lo, hi = plsc.unpack(u + v,
                                             format=plsc.PackFormat.INTERLEAVED)
                        ov[t, pl.ds(32 * half, 16)] = lo
                        ov[t, pl.ds(32 * half + 16, 16)] = hi

            pltpu.async_copy(ov, out_hbm.at[base + g], semos[p])
        return _

    lax.fori_loop(0, BPW // 2, outer, None)

    for p in range(2):
        pltpu.make_async_copy(outs[p], out_hbm.at[0], semos[p]).wait()


@jax.jit
def kernel(hour, weekday, day, month, E_hour, E_weekday, E_day, E_month):
    mesh = plsc.VectorSubcoreMesh(core_axis_name="c", subcore_axis_name="s")
    run = pl.kernel(
        _sc_body,
        out_type=jax.ShapeDtypeStruct((B, T, D), jnp.float32),
        mesh=mesh,
        compiler_params=pltpu.CompilerParams(use_tc_tiling_on_sc=True, needs_layout_passes=False),
        scratch_types=[
            pltpu.VMEM((24, D), jnp.float32),
            pltpu.VMEM((7, D), jnp.float32),
            pltpu.VMEM((32, D), jnp.float32),
            pltpu.VMEM((13, D), jnp.float32),
            pltpu.VMEM((N1 * 32,), jnp.int32),
            pltpu.VMEM((N2 * 32,), jnp.int32),
            pltpu.VMEM((T,), jnp.int32),
            pltpu.VMEM((T,), jnp.int32),
            pltpu.VMEM((T,), jnp.int32),
            pltpu.VMEM((T,), jnp.int32),
            pltpu.VMEM((T,), jnp.int32),
            pltpu.VMEM((T,), jnp.int32),
            pltpu.VMEM((T,), jnp.int32),
            pltpu.VMEM((T,), jnp.int32),
            pltpu.VMEM((T, D), jnp.float32),
            pltpu.VMEM((T, D), jnp.float32),
            pltpu.SemaphoreType.DMA,
            pltpu.SemaphoreType.DMA,
            pltpu.SemaphoreType.DMA,
            pltpu.SemaphoreType.DMA,
            pltpu.SemaphoreType.DMA,
        ],
    )
    return run(hour, weekday, day, month, E_hour, E_weekday, E_day, E_month)
